# Initial kernel scaffold; baseline (speedup 1.0000x reference)
import jax
import jax.numpy as jnp
from jax.experimental import pallas as pl


def kernel(X_in, ones):
    raise NotImplementedError("write your pallas kernel here")


# speedup vs baseline: 1.0048x; 1.0048x over previous
import functools

import jax
import jax.numpy as jnp
from jax import lax
from jax.experimental import pallas as pl
from jax.experimental.pallas import tpu as pltpu
from jax.experimental.pallas import tpu_sc as plsc

DEPTH = 1000
BATCH = 16384

NC = 2
NS = 16
L = 16
NW = NC * NS
BPW = BATCH // NW
WPW = BPW * DEPTH

ZW = 32000
NZ = WPW // ZW
SCT = 128
NSCT = BPW // SCT

_mesh = plsc.VectorSubcoreMesh(core_axis_name="c", subcore_axis_name="s")


@functools.partial(
    pl.kernel,
    out_type=jax.ShapeDtypeStruct((BATCH * DEPTH,), jnp.float32),
    mesh=_mesh,
    scratch_types=[
        pltpu.VMEM((BPW,), jnp.int32),
        pltpu.VMEM((ZW,), jnp.float32),
        pltpu.VMEM((SCT,), jnp.float32),
        [pltpu.VMEM((SCT,), jnp.int32) for _ in range(NSCT)],
        pltpu.SemaphoreType.DMA,
        pltpu.SemaphoreType.DMA,
    ],
)
def _onehot_flat(x_hbm, out_hbm, idx_v, zero_v, one_v, pos_vs, sem_z, sem_s):
    wid = lax.axis_index("s") * NC + lax.axis_index("c")
    base = wid * BPW

    pltpu.sync_copy(x_hbm.at[pl.ds(base, BPW)], idx_v)

    zeros16 = jnp.zeros((L,), jnp.float32)
    ones16 = jnp.ones((L,), jnp.float32)

    def zero_body(i, _):
        off = i * (8 * L)
        for j in range(8):
            zero_v[pl.ds(off + j * L, L)] = zeros16
        return 0

    lax.fori_loop(0, ZW // (8 * L), zero_body, 0)
    for j in range(SCT // L):
        one_v[pl.ds(j * L, L)] = ones16

    lane = lax.iota(jnp.int32, L)
    for m in range(BPW // L):
        idxs = idx_v[pl.ds(m * L, L)]
        rows = (base + m * L) + lane
        pos_vs[m // (SCT // L)][pl.ds((m % (SCT // L)) * L, L)] = (
            rows * DEPTH + idxs
        )

    fbase = base * DEPTH
    fills = [
        pltpu.async_copy(zero_v, out_hbm.at[pl.ds(fbase + i * ZW, ZW)], sem_z)
        for i in range(NZ)
    ]
    for f in fills:
        f.wait()

    scts = [
        pltpu.async_copy(one_v, out_hbm.at[pos_vs[k]], sem_s)
        for k in range(NSCT)
    ]
    for s in scts:
        s.wait()


def kernel(X_in, ones):
    del ones
    flat = _onehot_flat(X_in.astype(jnp.int32))
    return flat.reshape(BATCH, DEPTH)


# speedup vs baseline: 1.5861x; 1.5785x over previous
import functools

import jax
import jax.numpy as jnp
from jax import lax
from jax.experimental import pallas as pl
from jax.experimental.pallas import tpu as pltpu
from jax.experimental.pallas import tpu_sc as plsc

DEPTH = 1000
BATCH = 16384

NC = 2
NS = 16
L = 16
NW = NC * NS
BPW = BATCH // NW
CH = 32
NCH = BPW // CH

_mesh = plsc.VectorSubcoreMesh(core_axis_name="c", subcore_axis_name="s")


@functools.partial(
    pl.kernel,
    out_type=jax.ShapeDtypeStruct((BATCH, DEPTH), jnp.float32),
    mesh=_mesh,
    scratch_types=[
        pltpu.VMEM((BPW,), jnp.int32),
        pltpu.VMEM((CH, DEPTH), jnp.float32),
        pltpu.VMEM((CH, DEPTH), jnp.float32),
        pltpu.SemaphoreType.DMA,
        pltpu.SemaphoreType.DMA,
    ],
)
def _onehot_sc(x_hbm, out_hbm, idx_v, buf_a, buf_b, sem_a, sem_b):
    wid = lax.axis_index("s") * NC + lax.axis_index("c")
    base = wid * BPW

    pltpu.sync_copy(x_hbm.at[pl.ds(base, BPW)], idx_v)

    zeros16 = jnp.zeros((L,), jnp.float32)
    ones16 = jnp.ones((L,), jnp.float32)
    lane = lax.iota(jnp.int32, L)

    bufs = (buf_a, buf_b)
    sems = (sem_a, sem_b)

    def zero_rows(i, _):
        for o in list(range(0, DEPTH - L, L)) + [DEPTH - L]:
            buf_a[i, pl.ds(o, L)] = zeros16
            buf_b[i, pl.ds(o, L)] = zeros16
        return 0

    lax.fori_loop(0, CH, zero_rows, 0)

    def put(buf, c, set_one):
        for m in range(CH // L):
            idxs = idx_v[pl.ds(c * CH + m * L, L)]
            for r in range(L):
                cidx = idxs[r]
                aligned = (cidx // L) * L
                if set_one:
                    strip = jnp.where(
                        lane == cidx - aligned, 1.0, 0.0
                    ).astype(jnp.float32)
                else:
                    strip = zeros16
                buf[m * L + r, pl.ds(aligned, L)] = strip

    handles = {}
    for c in range(NCH):
        b = c % 2
        buf, sem = bufs[b], sems[b]
        if c >= 2:
            handles[c - 2].wait()
            put(buf, c - 2, set_one=False)
        put(buf, c, set_one=True)
        handles[c] = pltpu.async_copy(
            buf, out_hbm.at[pl.ds(base + c * CH, CH), :], sem
        )
    handles[NCH - 2].wait()
    handles[NCH - 1].wait()


def kernel(X_in, ones):
    del ones
    return _onehot_sc(X_in.astype(jnp.int32))


# speedup vs baseline: 3.8702x; 2.4401x over previous
import functools

import jax
import jax.numpy as jnp
from jax import lax
from jax.experimental import pallas as pl
from jax.experimental.pallas import tpu as pltpu
from jax.experimental.pallas import tpu_sc as plsc

DEPTH = 1000
BATCH = 16384

NC = 2
NS = 16
L = 16
NW = NC * NS
SPW = BATCH // NW
CR = 40
NCH = DEPTH // CR

_mesh = plsc.VectorSubcoreMesh(core_axis_name="c", subcore_axis_name="s")


@functools.partial(
    pl.kernel,
    out_type=jax.ShapeDtypeStruct((DEPTH, BATCH), jnp.float32),
    mesh=_mesh,
    scratch_types=[
        pltpu.VMEM((SPW,), jnp.int32),
        pltpu.VMEM((CR, SPW), jnp.float32),
        pltpu.VMEM((CR, SPW), jnp.float32),
        pltpu.SemaphoreType.DMA,
        pltpu.SemaphoreType.DMA,
    ],
    compiler_params=pltpu.CompilerParams(needs_layout_passes=False),
)
def _onehot_t_sc(x_hbm, out_hbm, idx_v, buf_a, buf_b, sem_a, sem_b):
    wid = lax.axis_index("s") * NC + lax.axis_index("c")
    base = wid * SPW

    pltpu.sync_copy(x_hbm.at[pl.ds(base, SPW)], idx_v)

    zeros16 = jnp.zeros((L,), jnp.float32)
    ones16 = jnp.ones((L,), jnp.float32)
    lane = lax.iota(jnp.int32, L)

    bufs = (buf_a, buf_b)
    sems = (sem_a, sem_b)

    def zero_rows(i, _):
        for o in range(0, SPW, L):
            buf_a[i, pl.ds(o, L)] = zeros16
            buf_b[i, pl.ds(o, L)] = zeros16
        return 0

    lax.fori_loop(0, CR, zero_rows, 0)

    def put(buf, row0, vals):
        for g in range(SPW // L):
            idxs = idx_v[pl.ds(g * L, L)]
            rows = idxs - row0
            mask = (rows >= 0) & (rows < CR)
            rows = jnp.clip(rows, 0, CR - 1)
            plsc.store_scatter(buf, [rows, g * L + lane], vals, mask=mask)

    def fire(buf, sem, c):
        put(buf, c * CR, ones16)
        pltpu.async_copy(
            buf, out_hbm.at[pl.ds(c * CR, CR), pl.ds(base, SPW)], sem
        )

    def drain_one(buf, sem):
        pltpu.make_async_copy(
            buf, out_hbm.at[pl.ds(0, CR), pl.ds(base, SPW)], sem
        ).wait()

    fire(buf_a, sem_a, 0)
    fire(buf_b, sem_b, 1)

    def step(c, _):
        for b in range(2):
            @pl.when(c % 2 == b)
            def _():
                buf, sem = bufs[b], sems[b]
                drain_one(buf, sem)
                put(buf, (c - 2) * CR, zeros16)
                fire(buf, sem, c)
        return 0

    lax.fori_loop(2, NCH, step, 0)
    drain_one(buf_a, sem_a)
    drain_one(buf_b, sem_b)


def kernel(X_in, ones):
    del ones
    return _onehot_t_sc(X_in.astype(jnp.int32)).T


# speedup vs baseline: 4.2135x; 1.0887x over previous
import functools

import jax
import jax.numpy as jnp
from jax import lax
from jax.experimental import pallas as pl
from jax.experimental.pallas import tpu as pltpu
from jax.experimental.pallas import tpu_sc as plsc

DEPTH = 1000
BATCH = 16384

NC = 2
NS = 16
L = 16
NW = NC * NS
SPW = BATCH // NW
CR = 40
NCH = DEPTH // CR

_mesh = plsc.VectorSubcoreMesh(core_axis_name="c", subcore_axis_name="s")


@functools.partial(
    pl.kernel,
    out_type=jax.ShapeDtypeStruct((DEPTH, BATCH), jnp.float32),
    mesh=_mesh,
    scratch_types=[
        pltpu.VMEM((SPW,), jnp.int32),
        pltpu.VMEM((CR, SPW), jnp.float32),
        pltpu.VMEM((CR, SPW), jnp.float32),
        pltpu.SemaphoreType.DMA,
        pltpu.SemaphoreType.DMA,
    ],
    compiler_params=pltpu.CompilerParams(needs_layout_passes=False),
)
def _onehot_t_sc(x_hbm, out_hbm, idx_v, buf_a, buf_b, sem_a, sem_b):
    wid = lax.axis_index("s") * NC + lax.axis_index("c")
    base = wid * SPW

    pltpu.sync_copy(x_hbm.at[pl.ds(base, SPW)], idx_v)

    zeros16 = jnp.zeros((L,), jnp.float32)
    ones16 = jnp.ones((L,), jnp.float32)
    lane = lax.iota(jnp.int32, L)

    bufs = (buf_a, buf_b)
    sems = (sem_a, sem_b)

    def zero_buf(buf):
        def zero_rows(i, _):
            for o in range(0, SPW, L):
                buf[i, pl.ds(o, L)] = zeros16
            return 0

        lax.fori_loop(0, CR, zero_rows, 0)

    idx_vecs = [idx_v[pl.ds(g * L, L)] for g in range(SPW // L)]
    col_vecs = [g * L + lane for g in range(SPW // L)]

    def put(buf, row0, vals):
        for g in range(SPW // L):
            rows = (idx_vecs[g] - row0).astype(jnp.uint32)
            mask = rows < CR
            rows = jnp.minimum(rows, CR - 1).astype(jnp.int32)
            plsc.store_scatter(buf, [rows, col_vecs[g]], vals, mask=mask)

    def fire(buf, sem, c):
        put(buf, c * CR, ones16)
        pltpu.async_copy(
            buf, out_hbm.at[pl.ds(c * CR, CR), pl.ds(base, SPW)], sem
        )

    def drain_one(buf, sem):
        pltpu.make_async_copy(
            buf, out_hbm.at[pl.ds(0, CR), pl.ds(base, SPW)], sem
        ).wait()

    zero_buf(buf_a)
    fire(buf_a, sem_a, 0)
    zero_buf(buf_b)
    fire(buf_b, sem_b, 1)

    def step(c, _):
        for b in range(2):
            @pl.when(c % 2 == b)
            def _():
                buf, sem = bufs[b], sems[b]
                drain_one(buf, sem)
                put(buf, (c - 2) * CR, zeros16)
                fire(buf, sem, c)
        return 0

    lax.fori_loop(2, NCH, step, 0)
    drain_one(buf_a, sem_a)
    drain_one(buf_b, sem_b)


def kernel(X_in, ones):
    del ones
    return _onehot_t_sc(X_in.astype(jnp.int32)).T


# speedup vs baseline: 4.3223x; 1.0258x over previous
import functools

import jax
import jax.numpy as jnp
from jax import lax
from jax.experimental import pallas as pl
from jax.experimental.pallas import tpu as pltpu
from jax.experimental.pallas import tpu_sc as plsc

DEPTH = 1000
BATCH = 16384

NC = 2
NS = 16
L = 16
NW = NC * NS
SPW = BATCH // NW
CR = 40
NCH = DEPTH // CR

_mesh = plsc.VectorSubcoreMesh(core_axis_name="c", subcore_axis_name="s")


@functools.partial(
    pl.kernel,
    out_type=jax.ShapeDtypeStruct((DEPTH, BATCH), jnp.float32),
    mesh=_mesh,
    scratch_types=[
        pltpu.VMEM((SPW,), jnp.int32),
        pltpu.VMEM((CR, SPW), jnp.float32),
        pltpu.VMEM((CR, SPW), jnp.float32),
        pltpu.SemaphoreType.DMA,
        pltpu.SemaphoreType.DMA,
    ],
    compiler_params=pltpu.CompilerParams(needs_layout_passes=False),
)
def _onehot_t_sc(x_hbm, out_hbm, idx_v, buf_a, buf_b, sem_a, sem_b):
    wid = lax.axis_index("s") * NC + lax.axis_index("c")
    base = wid * SPW

    pltpu.sync_copy(x_hbm.at[pl.ds(base, SPW)], idx_v)

    zeros16 = jnp.zeros((L,), jnp.float32)
    ones16 = jnp.ones((L,), jnp.float32)
    lane = lax.iota(jnp.int32, L)

    bufs = (buf_a, buf_b)
    sems = (sem_a, sem_b)

    def zero_buf(buf):
        def zero_rows(i, _):
            for o in range(0, SPW, L):
                buf[i, pl.ds(o, L)] = zeros16
            return 0

        lax.fori_loop(0, CR, zero_rows, 0)

    def put(buf, row0, vals):
        def group(g, _):
            idxs = idx_v[pl.ds(g * L, L)]
            rows = (idxs - row0).astype(jnp.uint32)
            mask = rows < CR
            rows = jnp.minimum(rows, CR - 1).astype(jnp.int32)
            plsc.store_scatter(buf, [rows, g * L + lane], vals, mask=mask)
            return 0

        lax.fori_loop(0, SPW // L, group, 0)

    def fire(buf, sem, c):
        put(buf, c * CR, ones16)
        pltpu.async_copy(
            buf, out_hbm.at[pl.ds(c * CR, CR), pl.ds(base, SPW)], sem
        )

    def drain_one(buf, sem):
        pltpu.make_async_copy(
            buf, out_hbm.at[pl.ds(0, CR), pl.ds(base, SPW)], sem
        ).wait()

    zero_buf(buf_a)
    fire(buf_a, sem_a, 0)
    zero_buf(buf_b)
    fire(buf_b, sem_b, 1)

    def step(c, _):
        for b in range(2):
            @pl.when(c % 2 == b)
            def _():
                buf, sem = bufs[b], sems[b]
                drain_one(buf, sem)
                put(buf, (c - 2) * CR, zeros16)
                fire(buf, sem, c)
        return 0

    lax.fori_loop(2, NCH, step, 0)
    drain_one(buf_a, sem_a)
    drain_one(buf_b, sem_b)


def kernel(X_in, ones):
    del ones
    return _onehot_t_sc(X_in.astype(jnp.int32)).T
